# Initial kernel scaffold; baseline (speedup 1.0000x reference)
import jax
import jax.numpy as jnp
from jax.experimental import pallas as pl


def kernel(voxel_features, coors):
    raise NotImplementedError("write your pallas kernel here")


# speedup vs baseline: 1.1645x; 1.1645x over previous
import functools

import jax
import jax.numpy as jnp
from jax import lax
from jax.experimental import pallas as pl
from jax.experimental.pallas import tpu as pltpu
from jax.experimental.pallas import tpu_sc as plsc

C = 64
NXY = 512
NPIX = NXY * NXY
NV = 40000
NV_PAD = 40960
NW = 32
PIX_PER_TILE = NPIX // NW
ROW_CHUNK = 8000
N_CHUNKS = NV // ROW_CHUNK
PIX_CHUNK = 8192
L = 16


def _transpose_body(x_ref, o_ref):
    o_ref[...] = x_ref[...].T


def _tc_transpose(vf_pad):
    blk = 4096
    return pl.pallas_call(
        _transpose_body,
        grid=(NV_PAD // blk,),
        in_specs=[pl.BlockSpec((blk, C), lambda i: (i, 0))],
        out_specs=pl.BlockSpec((C, blk), lambda i: (0, i)),
        out_shape=jax.ShapeDtypeStruct((C, NV_PAD), jnp.float32),
    )(vf_pad)


_mesh = plsc.VectorSubcoreMesh(core_axis_name="c", subcore_axis_name="s")


@functools.partial(
    pl.kernel,
    mesh=_mesh,
    out_type=jax.ShapeDtypeStruct((NPIX,), jnp.int32),
    scratch_types=[
        pltpu.VMEM((ROW_CHUNK * 3,), jnp.int32),
        pltpu.VMEM((PIX_PER_TILE,), jnp.int32),
    ],
    compiler_params=pltpu.CompilerParams(needs_layout_passes=False),
)
def _sc_winner(coors_hbm, w_hbm, coors_v, w_v):
    wid = lax.axis_index("s") * 2 + lax.axis_index("c")
    base = wid * PIX_PER_TILE
    iota = lax.iota(jnp.int32, L)
    posy = iota * 3 + 1
    posx = iota * 3 + 2
    neg1 = jnp.full((L,), -1, jnp.int32)

    def _init(i, carry):
        w_v[pl.ds(i * L, L)] = neg1
        return carry

    lax.fori_loop(0, PIX_PER_TILE // L, _init, 0)

    for ci in range(N_CHUNKS):
        pltpu.sync_copy(
            coors_hbm.at[pl.ds(ci * ROW_CHUNK * 3, ROW_CHUNK * 3)], coors_v
        )
        nbase = ci * ROW_CHUNK

        def _vreg(g, carry, nbase=nbase):
            b3 = g * (3 * L)
            y = plsc.load_gather(coors_v, [posy + b3])
            x = plsc.load_gather(coors_v, [posx + b3])
            local = y * NXY + x - base
            nvec = nbase + g * L + iota
            mask = (local >= 0) & (local < PIX_PER_TILE)

            def _cond(m):
                return jnp.max(m.astype(jnp.int32)) > 0

            def _body(m):
                plsc.store_scatter(w_v, [local], nvec, mask=m)
                got = plsc.load_gather(w_v, [local], mask=m)
                return m & (got < nvec)

            lax.while_loop(_cond, _body, mask)
            return carry

        lax.fori_loop(0, ROW_CHUNK // L, _vreg, 0)

    def _fix(i, carry):
        v = w_v[pl.ds(i * L, L)]
        w_v[pl.ds(i * L, L)] = jnp.where(v < 0, jnp.int32(NV), v)
        return carry

    lax.fori_loop(0, PIX_PER_TILE // L, _fix, 0)
    pltpu.sync_copy(w_v, w_hbm.at[pl.ds(base, PIX_PER_TILE)])


@functools.partial(
    pl.kernel,
    mesh=_mesh,
    out_type=jax.ShapeDtypeStruct((C, NPIX), jnp.float32),
    scratch_types=[
        pltpu.VMEM((NV_PAD,), jnp.float32),
        pltpu.VMEM((NV_PAD,), jnp.float32),
        pltpu.VMEM((PIX_CHUNK,), jnp.int32),
        pltpu.VMEM((PIX_CHUNK,), jnp.float32),
        pltpu.VMEM((PIX_CHUNK,), jnp.float32),
    ],
    compiler_params=pltpu.CompilerParams(needs_layout_passes=False),
)
def _sc_gather(vft_hbm, w_hbm, out_hbm, t0, t1, wv, o0, o1):
    wid = lax.axis_index("s") * 2 + lax.axis_index("c")
    c0 = wid * 2
    pltpu.sync_copy(vft_hbm.at[c0], t0)
    pltpu.sync_copy(vft_hbm.at[c0 + 1], t1)

    def _chunk(p, carry):
        pltpu.sync_copy(w_hbm.at[pl.ds(p * PIX_CHUNK, PIX_CHUNK)], wv)

        def _vreg(g, inner):
            wvec = wv[pl.ds(g * L, L)]
            o0[pl.ds(g * L, L)] = plsc.load_gather(t0, [wvec])
            o1[pl.ds(g * L, L)] = plsc.load_gather(t1, [wvec])
            return inner

        lax.fori_loop(0, PIX_CHUNK // L, _vreg, 0)
        pltpu.sync_copy(o0, out_hbm.at[c0, pl.ds(p * PIX_CHUNK, PIX_CHUNK)])
        pltpu.sync_copy(o1, out_hbm.at[c0 + 1, pl.ds(p * PIX_CHUNK, PIX_CHUNK)])
        return carry

    lax.fori_loop(0, NPIX // PIX_CHUNK, _chunk, 0)


def kernel(voxel_features, coors):
    vf_pad = jnp.pad(voxel_features, ((0, NV_PAD - NV), (0, 0)))
    vft = _tc_transpose(vf_pad)
    w = _sc_winner(coors.reshape(-1))
    canvas = _sc_gather(vft, w)
    return canvas.reshape(1, C, NXY, NXY)
